# Initial kernel scaffold; baseline (speedup 1.0000x reference)
import jax
import jax.numpy as jnp
from jax.experimental import pallas as pl


def kernel(x_sub, x_agr, x_urb, ei_down, ei_agr2sub, ei_sub2agr, ei_urb2sub, ei_sub2urb, Wl, bl, Wr, Wf, bf):
    raise NotImplementedError("write your pallas kernel here")


# speedup vs baseline: 1.9889x; 1.9889x over previous
import functools

import jax
import jax.numpy as jnp
from jax import lax
from jax.experimental import pallas as pl
from jax.experimental.pallas import tpu as pltpu
from jax.experimental.pallas import tpu_sc as plsc

N = 10000
E = 160000
D = 128
D2 = 64
OUT = 8
NCORES = 2
NSUB = 16
NW = NCORES * NSUB
CHUNK = 128
EPAD = 163840
CPT = EPAD // (NW * CHUNK)
ACC_ROWS = 10240
ZROWS = ACC_ROWS // NSUB
OROWS = ACC_ROWS // NSUB
CNTW = 16

_mesh = plsc.VectorSubcoreMesh(core_axis_name="c", subcore_axis_name="s")


@functools.partial(
    pl.kernel,
    mesh=_mesh,
    out_type=jax.ShapeDtypeStruct((NCORES, ACC_ROWS, D2), jnp.float32),
    scratch_types=[
        pltpu.VMEM((2, CHUNK), jnp.int32),
        pltpu.VMEM((CHUNK, D2), jnp.float32),
        pltpu.VMEM((ZROWS, D2), jnp.float32),
        pltpu.VMEM_SHARED((ACC_ROWS, D2), jnp.float32),
        pltpu.SemaphoreType.DMA,
    ],
    compiler_params=pltpu.CompilerParams(use_tc_tiling_on_sc=False),
)
def _seg_sum_sc(x_hbm, src_hbm, dst_hbm, out_hbm, idx_v, rows_v, zbuf_v, acc_sh, sem):
    c = lax.axis_index("c")
    s = lax.axis_index("s")
    tid = c * NSUB + s

    def _zrow(r, _):
        def _zlane(j, _):
            zbuf_v[r, pl.ds(j * 16, 16)] = jnp.zeros((16,), jnp.float32)
            return 0
        return lax.fori_loop(0, D2 // 16, _zlane, 0)

    lax.fori_loop(0, ZROWS, _zrow, 0)
    pltpu.sync_copy(zbuf_v, acc_sh.at[pl.ds(s * ZROWS, ZROWS)])
    plsc.subcore_barrier()

    def _chunk(i, _):
        base = (tid * CPT + i) * CHUNK
        pltpu.sync_copy(src_hbm.at[pl.ds(base, CHUNK)], idx_v.at[0])
        pltpu.sync_copy(dst_hbm.at[pl.ds(base, CHUNK)], idx_v.at[1])
        pltpu.async_copy(x_hbm.at[idx_v.at[0]], rows_v, sem).wait()
        pltpu.sync_copy(rows_v, acc_sh.at[idx_v.at[1]], add=True)
        return 0

    lax.fori_loop(0, CPT, _chunk, 0)
    plsc.subcore_barrier()

    r0 = s * OROWS
    pltpu.sync_copy(acc_sh.at[pl.ds(r0, OROWS)], out_hbm.at[c].at[pl.ds(r0, OROWS)])


@functools.partial(
    pl.kernel,
    mesh=_mesh,
    out_type=jax.ShapeDtypeStruct((NCORES, ACC_ROWS, CNTW), jnp.float32),
    scratch_types=[
        pltpu.VMEM((2, CHUNK), jnp.int32),
        pltpu.VMEM((CHUNK, CNTW), jnp.float32),
        pltpu.VMEM((ZROWS, CNTW), jnp.float32),
        pltpu.VMEM_SHARED((ACC_ROWS, CNTW), jnp.float32),
    ],
    compiler_params=pltpu.CompilerParams(use_tc_tiling_on_sc=False),
)
def _seg_cnt_sc(dst_hbm, out_hbm, idx_v, ones_v, zbuf_v, acc_sh):
    c = lax.axis_index("c")
    s = lax.axis_index("s")
    tid = c * NSUB + s

    def _orow(r, _):
        ones_v[r, pl.ds(0, 16)] = jnp.ones((16,), jnp.float32)
        return 0

    lax.fori_loop(0, CHUNK, _orow, 0)

    def _zrow(r, _):
        zbuf_v[r, pl.ds(0, 16)] = jnp.zeros((16,), jnp.float32)
        return 0

    lax.fori_loop(0, ZROWS, _zrow, 0)
    pltpu.sync_copy(zbuf_v, acc_sh.at[pl.ds(s * ZROWS, ZROWS)])
    plsc.subcore_barrier()

    def _chunk(i, _):
        base = (tid * CPT + i) * CHUNK
        pltpu.sync_copy(dst_hbm.at[pl.ds(base, CHUNK)], idx_v.at[1])
        pltpu.sync_copy(ones_v, acc_sh.at[idx_v.at[1]], add=True)
        return 0

    lax.fori_loop(0, CPT, _chunk, 0)
    plsc.subcore_barrier()

    r0 = s * OROWS
    pltpu.sync_copy(acc_sh.at[pl.ds(r0, OROWS)], out_hbm.at[c].at[pl.ds(r0, OROWS)])


def _seg_sum_full(x2, src_lo, src_hi, dst):
    lo = _seg_sum_sc(x2, src_lo, dst)
    hi = _seg_sum_sc(x2, src_hi, dst)
    return lo, hi


BLK = 1000


def _halfdot(lo, hi, w_ref):
    a = jnp.dot(lo, w_ref[0:D2, :], preferred_element_type=jnp.float32)
    return a + jnp.dot(hi, w_ref[D2:D, :], preferred_element_type=jnp.float32)


def _layer_body(is_mean, sdl_ref, sdh_ref, sal_ref, sah_ref, sul_ref, suh_ref,
                h_ref, cnt_ref, w0_ref, w1_ref, w3_ref, wr_ref, b_ref, out_ref):
    sdl = sdl_ref[0] + sdl_ref[1]
    sdh = sdh_ref[0] + sdh_ref[1]
    if is_mean:
        inv = 1.0 / jnp.maximum(cnt_ref[0, :, 0:1] + cnt_ref[1, :, 0:1], 1.0)
        sdl = sdl * inv
        sdh = sdh * inv
    acc = _halfdot(sdl, sdh, w0_ref)
    acc = acc + _halfdot(sal_ref[0] + sal_ref[1], sah_ref[0] + sah_ref[1], w1_ref)
    acc = acc + _halfdot(sul_ref[0] + sul_ref[1], suh_ref[0] + suh_ref[1], w3_ref)
    acc = acc + jnp.dot(h_ref[...], wr_ref[...], preferred_element_type=jnp.float32)
    out_ref[...] = jnp.maximum(acc + b_ref[...], 0.0)


def _make_layer_tc(is_mean):
    half = pl.BlockSpec((NCORES, BLK, D2), lambda i: (0, i, 0))
    full = pl.BlockSpec((D, D), lambda i: (0, 0))
    return pl.pallas_call(
        functools.partial(_layer_body, is_mean),
        grid=(N // BLK,),
        in_specs=[
            half, half, half, half, half, half,
            pl.BlockSpec((BLK, D), lambda i: (i, 0)),
            pl.BlockSpec((NCORES, BLK, CNTW), lambda i: (0, i, 0)),
            full, full, full, full,
            pl.BlockSpec((1, D), lambda i: (0, 0)),
        ],
        out_specs=pl.BlockSpec((BLK, D), lambda i: (i, 0)),
        out_shape=jax.ShapeDtypeStruct((N, D), jnp.float32),
    )


_layer_tc_sum = _make_layer_tc(False)
_layer_tc_mean = _make_layer_tc(True)


def _final_body(h_ref, wf_ref, bf_ref, out_ref):
    logits = jnp.dot(h_ref[...], wf_ref[...],
                     preferred_element_type=jnp.float32) + bf_ref[...]
    m = jnp.max(logits, axis=1, keepdims=True)
    e = jnp.exp(logits - m)
    out_ref[...] = e / jnp.sum(e, axis=1, keepdims=True)


_final_tc = pl.pallas_call(
    _final_body,
    grid=(N // BLK,),
    in_specs=[
        pl.BlockSpec((BLK, D), lambda i: (i, 0)),
        pl.BlockSpec((D, OUT), lambda i: (0, 0)),
        pl.BlockSpec((1, OUT), lambda i: (0, 0)),
    ],
    out_specs=pl.BlockSpec((BLK, OUT), lambda i: (i, 0)),
    out_shape=jax.ShapeDtypeStruct((N, OUT), jnp.float32),
)


def _pad_edges(ei):
    src = jnp.concatenate(
        [ei[0].astype(jnp.int32), jnp.zeros((EPAD - E,), jnp.int32)])
    dst = jnp.concatenate(
        [ei[1].astype(jnp.int32), jnp.full((EPAD - E,), N, jnp.int32)])
    return 2 * src, 2 * src + 1, dst


def kernel(x_sub, x_agr, x_urb, ei_down, ei_agr2sub, ei_sub2agr, ei_urb2sub,
           ei_sub2urb, Wl, bl, Wr, Wf, bf):
    srcd_lo, srcd_hi, dst_d = _pad_edges(ei_down)
    srca_lo, srca_hi, dst_a = _pad_edges(ei_agr2sub)
    srcu_lo, srcu_hi, dst_u = _pad_edges(ei_urb2sub)

    sal, sah = _seg_sum_full(x_agr.reshape(2 * N, D2), srca_lo, srca_hi, dst_a)
    sul, suh = _seg_sum_full(x_urb.reshape(2 * N, D2), srcu_lo, srcu_hi, dst_u)
    cnt = _seg_cnt_sc(dst_d)

    h = x_sub
    for l in range(3):
        sdl, sdh = _seg_sum_full(h.reshape(2 * N, D2), srcd_lo, srcd_hi, dst_d)
        wr = Wr[l, 0] + Wr[l, 1] + Wr[l, 3]
        b = (bl[l, 0] + bl[l, 1] + bl[l, 3]).reshape(1, D)
        layer = _layer_tc_sum if l == 0 else _layer_tc_mean
        h = layer(sdl, sdh, sal, sah, sul, suh, h, cnt,
                  Wl[l, 0], Wl[l, 1], Wl[l, 3], wr, b)

    return _final_tc(h, Wf, bf.reshape(1, OUT))


# speedup vs baseline: 2.4476x; 1.2306x over previous
import functools

import jax
import jax.numpy as jnp
from jax import lax
from jax.experimental import pallas as pl
from jax.experimental.pallas import tpu as pltpu
from jax.experimental.pallas import tpu_sc as plsc

N = 10000
E = 160000
D = 128
D2 = 64
OUT = 8
NCORES = 2
NSUB = 16
NW = NCORES * NSUB
CHUNK = 128
EPAD = 163840
NCHUNK = EPAD // CHUNK
CPT = NCHUNK // NW
ACC_ROWS = 10240
ZROWS = ACC_ROWS // NSUB
QZ = ZROWS // 4
CNTW = 16
NBUF = 8

_mesh = plsc.VectorSubcoreMesh(core_axis_name="c", subcore_axis_name="s")
_sc_params = pltpu.CompilerParams(use_tc_tiling_on_sc=False)


@functools.partial(
    pl.kernel,
    mesh=_mesh,
    out_type=jax.ShapeDtypeStruct((NCORES, ACC_ROWS, D2), jnp.float32),
    scratch_types=[
        pltpu.VMEM((CPT, CHUNK), jnp.int32),
        pltpu.VMEM((CPT, CHUNK), jnp.int32),
        pltpu.VMEM((NBUF, CHUNK, D2), jnp.float32),
        pltpu.VMEM((QZ, D2), jnp.float32),
        pltpu.VMEM_SHARED((ACC_ROWS, D2), jnp.float32),
        pltpu.SemaphoreType.DMA,
        pltpu.SemaphoreType.DMA,
    ],
    compiler_params=_sc_params,
)
def _seg_sum_sc(x_hbm, src_hbm, dst_hbm, tok_hbm, out_hbm,
                src_v, dst_v, bufs_v, zbuf_v, acc_sh, sem_g, sem_s):
    del tok_hbm
    c = lax.axis_index("c")
    s = lax.axis_index("s")
    tid = c * NSUB + s

    pltpu.sync_copy(src_hbm.at[pl.ds(tid * CPT, CPT)], src_v)
    pltpu.sync_copy(dst_hbm.at[pl.ds(tid * CPT, CPT)], dst_v)

    def _zrow(r, _):
        def _zlane(j, _):
            zbuf_v[r, pl.ds(j * 16, 16)] = jnp.zeros((16,), jnp.float32)
            return 0
        return lax.fori_loop(0, D2 // 16, _zlane, 0)

    lax.fori_loop(0, QZ, _zrow, 0)
    for q in range(4):
        pltpu.sync_copy(zbuf_v, acc_sh.at[pl.ds(s * ZROWS + q * QZ, QZ)])
    plsc.subcore_barrier()

    def _quad(i, _):
        j0 = i * NBUF
        for half in range(2):
            gs = []
            for u in range(4):
                b = half * 4 + u
                j = j0 + b
                gs.append(pltpu.async_copy(
                    x_hbm.at[src_v.at[j]], bufs_v.at[b], sem_g))
            for g in gs:
                g.wait()
            for u in range(4):
                b = half * 4 + u
                j = j0 + b
                pltpu.async_copy(
                    bufs_v.at[b], acc_sh.at[dst_v.at[j]], sem_s, add=True)
        for b in range(NBUF):
            pltpu.make_async_copy(
                bufs_v.at[b], acc_sh.at[dst_v.at[j0 + b]], sem_s).wait()
        return 0

    lax.fori_loop(0, CPT // NBUF, _quad, 0)
    plsc.subcore_barrier()

    r0 = s * ZROWS
    pltpu.sync_copy(acc_sh.at[pl.ds(r0, ZROWS)], out_hbm.at[c].at[pl.ds(r0, ZROWS)])


@functools.partial(
    pl.kernel,
    mesh=_mesh,
    out_type=jax.ShapeDtypeStruct((NCORES, ACC_ROWS, CNTW), jnp.float32),
    scratch_types=[
        pltpu.VMEM((CPT, CHUNK), jnp.int32),
        pltpu.VMEM((CHUNK, CNTW), jnp.float32),
        pltpu.VMEM((ZROWS, CNTW), jnp.float32),
        pltpu.VMEM_SHARED((ACC_ROWS, CNTW), jnp.float32),
        pltpu.SemaphoreType.DMA,
    ],
    compiler_params=_sc_params,
)
def _seg_cnt_sc(dst_hbm, tok_hbm, out_hbm, dst_v, ones_v, zbuf_v, acc_sh, sem_s):
    del tok_hbm
    c = lax.axis_index("c")
    s = lax.axis_index("s")
    tid = c * NSUB + s

    pltpu.sync_copy(dst_hbm.at[pl.ds(tid * CPT, CPT)], dst_v)

    def _orow(r, _):
        ones_v[r, pl.ds(0, 16)] = jnp.ones((16,), jnp.float32)
        return 0

    lax.fori_loop(0, CHUNK, _orow, 0)

    def _zrow(r, _):
        zbuf_v[r, pl.ds(0, 16)] = jnp.zeros((16,), jnp.float32)
        return 0

    lax.fori_loop(0, ZROWS, _zrow, 0)
    pltpu.sync_copy(zbuf_v, acc_sh.at[pl.ds(s * ZROWS, ZROWS)])
    plsc.subcore_barrier()

    def _fire(i, _):
        for u in range(4):
            pltpu.async_copy(ones_v, acc_sh.at[dst_v.at[i * 4 + u]], sem_s,
                             add=True)
        return 0

    lax.fori_loop(0, CPT // 4, _fire, 0)

    def _drain(i, _):
        for u in range(4):
            pltpu.make_async_copy(ones_v, acc_sh.at[dst_v.at[0]], sem_s).wait()
        return 0

    lax.fori_loop(0, CPT // 4, _drain, 0)
    plsc.subcore_barrier()

    r0 = s * ZROWS
    pltpu.sync_copy(acc_sh.at[pl.ds(r0, ZROWS)], out_hbm.at[c].at[pl.ds(r0, ZROWS)])


def _token(arr):
    return lax.slice(arr.reshape(-1), (0,), (8,))


BLK = 1000


def _halfdot(lo, hi, w_ref):
    a = jnp.dot(lo, w_ref[0:D2, :], preferred_element_type=jnp.float32)
    return a + jnp.dot(hi, w_ref[D2:D, :], preferred_element_type=jnp.float32)


def _layer_body(is_mean, sd_ref, sa_ref, su_ref, h_ref, cnt_ref,
                w0_ref, w1_ref, w3_ref, wr_ref, b_ref, out_ref):
    sdl = sd_ref[0, 0] + sd_ref[0, 1]
    sdh = sd_ref[1, 0] + sd_ref[1, 1]
    if is_mean:
        inv = 1.0 / jnp.maximum(cnt_ref[0, :, 0:1] + cnt_ref[1, :, 0:1], 1.0)
        sdl = sdl * inv
        sdh = sdh * inv
    acc = _halfdot(sdl, sdh, w0_ref)
    acc = acc + _halfdot(sa_ref[0, 0] + sa_ref[0, 1], sa_ref[1, 0] + sa_ref[1, 1], w1_ref)
    acc = acc + _halfdot(su_ref[0, 0] + su_ref[0, 1], su_ref[1, 0] + su_ref[1, 1], w3_ref)
    acc = acc + jnp.dot(h_ref[...], wr_ref[...], preferred_element_type=jnp.float32)
    out_ref[...] = jnp.maximum(acc + b_ref[...], 0.0)


def _make_layer_tc(is_mean):
    part = pl.BlockSpec((2, NCORES, BLK, D2), lambda i: (0, 0, i, 0))
    full = pl.BlockSpec((D, D), lambda i: (0, 0))
    return pl.pallas_call(
        functools.partial(_layer_body, is_mean),
        grid=(N // BLK,),
        in_specs=[
            part, part, part,
            pl.BlockSpec((BLK, D), lambda i: (i, 0)),
            pl.BlockSpec((NCORES, BLK, CNTW), lambda i: (0, i, 0)),
            full, full, full, full,
            pl.BlockSpec((1, D), lambda i: (0, 0)),
        ],
        out_specs=pl.BlockSpec((BLK, D), lambda i: (i, 0)),
        out_shape=jax.ShapeDtypeStruct((N, D), jnp.float32),
    )


_layer_tc_sum = _make_layer_tc(False)
_layer_tc_mean = _make_layer_tc(True)


def _final_body(h_ref, wf_ref, bf_ref, out_ref):
    logits = jnp.dot(h_ref[...], wf_ref[...],
                     preferred_element_type=jnp.float32) + bf_ref[...]
    m = jnp.max(logits, axis=1, keepdims=True)
    e = jnp.exp(logits - m)
    out_ref[...] = e / jnp.sum(e, axis=1, keepdims=True)


_final_tc = pl.pallas_call(
    _final_body,
    grid=(N // BLK,),
    in_specs=[
        pl.BlockSpec((BLK, D), lambda i: (i, 0)),
        pl.BlockSpec((D, OUT), lambda i: (0, 0)),
        pl.BlockSpec((1, OUT), lambda i: (0, 0)),
    ],
    out_specs=pl.BlockSpec((BLK, OUT), lambda i: (i, 0)),
    out_shape=jax.ShapeDtypeStruct((N, OUT), jnp.float32),
)


def _pad_edges(ei):
    src = jnp.concatenate(
        [ei[0].astype(jnp.int32), jnp.zeros((EPAD - E,), jnp.int32)])
    dst = jnp.concatenate(
        [ei[1].astype(jnp.int32), jnp.full((EPAD - E,), N, jnp.int32)])
    return ((2 * src).reshape(NCHUNK, CHUNK),
            (2 * src + 1).reshape(NCHUNK, CHUNK),
            dst.reshape(NCHUNK, CHUNK))


def kernel(x_sub, x_agr, x_urb, ei_down, ei_agr2sub, ei_sub2agr, ei_urb2sub,
           ei_sub2urb, Wl, bl, Wr, Wf, bf):
    srcd_lo, srcd_hi, dst_d = _pad_edges(ei_down)
    srca_lo, srca_hi, dst_a = _pad_edges(ei_agr2sub)
    srcu_lo, srcu_hi, dst_u = _pad_edges(ei_urb2sub)

    xa2 = x_agr.reshape(2 * N, D2)
    xu2 = x_urb.reshape(2 * N, D2)
    sa_lo = _seg_sum_sc(xa2, srca_lo, dst_a, jnp.zeros((8,), jnp.float32))
    sa_hi = _seg_sum_sc(xa2, srca_hi, dst_a, _token(sa_lo))
    su_lo = _seg_sum_sc(xu2, srcu_lo, dst_u, _token(sa_hi))
    su_hi = _seg_sum_sc(xu2, srcu_hi, dst_u, _token(su_lo))
    sa = jnp.stack([sa_lo, sa_hi])
    su = jnp.stack([su_lo, su_hi])
    cnt = _seg_cnt_sc(dst_d, _token(su_hi))

    h = x_sub
    tok = _token(cnt)
    for l in range(3):
        h2 = h.reshape(2 * N, D2)
        sd_lo = _seg_sum_sc(h2, srcd_lo, dst_d, tok)
        sd_hi = _seg_sum_sc(h2, srcd_hi, dst_d, _token(sd_lo))
        sd = jnp.stack([sd_lo, sd_hi])
        tok = _token(sd_hi)
        wr = Wr[l, 0] + Wr[l, 1] + Wr[l, 3]
        b = (bl[l, 0] + bl[l, 1] + bl[l, 3]).reshape(1, D)
        layer = _layer_tc_sum if l == 0 else _layer_tc_mean
        h = layer(sd, sa, su, h, cnt, Wl[l, 0], Wl[l, 1], Wl[l, 3], wr, b)

    return _final_tc(h, Wf, bf.reshape(1, OUT))


# speedup vs baseline: 2.7186x; 1.1107x over previous
import functools

import jax
import jax.numpy as jnp
from jax import lax
from jax.experimental import pallas as pl
from jax.experimental.pallas import tpu as pltpu
from jax.experimental.pallas import tpu_sc as plsc

N = 10000
E = 160000
D = 128
D2 = 64
OUT = 8
NCORES = 2
NSUB = 16
NW = NCORES * NSUB
CHUNK = 128
EPAD = 163840
NCHUNK = EPAD // CHUNK
CPT = NCHUNK // NW
ACC_ROWS = 10240
ZROWS = ACC_ROWS // NSUB
QZ = ZROWS // 4
NBUF = 8
AHEAD = 4

_mesh = plsc.VectorSubcoreMesh(core_axis_name="c", subcore_axis_name="s")
_sc_params = pltpu.CompilerParams(use_tc_tiling_on_sc=False)
_sc_params_nl = pltpu.CompilerParams(use_tc_tiling_on_sc=False,
                                     needs_layout_passes=False)


def _zero_vmem(ref, rows, width):
    def _zrow(r, _):
        def _zlane(j, _):
            ref[r, pl.ds(j * 16, 16)] = jnp.zeros((16,), jnp.float32)
            return 0
        return lax.fori_loop(0, width // 16, _zlane, 0)
    lax.fori_loop(0, rows, _zrow, 0)


def _zero_acc(s, zbuf_v, acc_sh):
    for q in range(4):
        pltpu.sync_copy(zbuf_v, acc_sh.at[pl.ds(s * ZROWS + q * QZ, QZ)])


def _accumulate(x_hbm, src_v, dst_v, bufs_v, acc_sh, sem_g, sem_s):
    for p in range(AHEAD):
        pltpu.async_copy(x_hbm.at[src_v.at[p]], bufs_v.at[p], sem_g)

    def _step(j, _):
        b = lax.rem(j, NBUF)
        pltpu.make_async_copy(x_hbm.at[src_v.at[0]], bufs_v.at[0], sem_g).wait()
        pltpu.async_copy(bufs_v.at[b], acc_sh.at[dst_v.at[j]], sem_s, add=True)

        @pl.when(j >= AHEAD)
        def _():
            pltpu.make_async_copy(
                bufs_v.at[0], acc_sh.at[dst_v.at[0]], sem_s).wait()

        @pl.when(j + AHEAD < CPT)
        def _():
            b2 = lax.rem(j + AHEAD, NBUF)
            pltpu.async_copy(x_hbm.at[src_v.at[j + AHEAD]], bufs_v.at[b2], sem_g)

        return 0

    lax.fori_loop(0, CPT, _step, 0)
    for p in range(AHEAD):
        pltpu.make_async_copy(bufs_v.at[0], acc_sh.at[dst_v.at[0]], sem_s).wait()


def _seg_phase(k, x_hbm, src_hbm, dst_hbm, out_hbm, src_v, dst_v, bufs_v,
               zbuf_v, acc_sh, sem_g, sem_s, c, s, tid, first):
    pltpu.sync_copy(src_hbm.at[pl.ds(tid * CPT, CPT)], src_v)
    pltpu.sync_copy(dst_hbm.at[pl.ds(tid * CPT, CPT)], dst_v)
    if not first:
        plsc.subcore_barrier()
    _zero_acc(s, zbuf_v, acc_sh)
    plsc.subcore_barrier()
    _accumulate(x_hbm, src_v, dst_v, bufs_v, acc_sh, sem_g, sem_s)
    plsc.subcore_barrier()
    r0 = s * ZROWS
    pltpu.sync_copy(acc_sh.at[pl.ds(r0, ZROWS)],
                    out_hbm.at[k].at[c].at[pl.ds(r0, ZROWS)])


@functools.partial(
    pl.kernel,
    mesh=_mesh,
    out_type=(
        jax.ShapeDtypeStruct((2, NCORES, ACC_ROWS, D2), jnp.float32),
        jax.ShapeDtypeStruct((2, NCORES, ACC_ROWS, D2), jnp.float32),
    ),
    scratch_types=[
        pltpu.VMEM((CPT, CHUNK), jnp.int32),
        pltpu.VMEM((CPT, CHUNK), jnp.int32),
        pltpu.VMEM((NBUF, CHUNK, D2), jnp.float32),
        pltpu.VMEM((QZ, D2), jnp.float32),
        pltpu.VMEM_SHARED((ACC_ROWS, D2), jnp.float32),
        pltpu.SemaphoreType.DMA,
        pltpu.SemaphoreType.DMA,
    ],
    compiler_params=_sc_params,
)
def _invariants_sc(xa_hbm, xu_hbm, srca_lo, srca_hi, dsta_hbm,
                   srcu_lo, srcu_hi, dstu_hbm,
                   sa_hbm, su_hbm,
                   src_v, dst_v, bufs_v, zbuf_v, acc_sh, sem_g, sem_s):
    c = lax.axis_index("c")
    s = lax.axis_index("s")
    tid = c * NSUB + s

    _zero_vmem(zbuf_v, QZ, D2)

    _seg_phase(0, xa_hbm, srca_lo, dsta_hbm, sa_hbm, src_v, dst_v, bufs_v,
               zbuf_v, acc_sh, sem_g, sem_s, c, s, tid, True)
    _seg_phase(1, xa_hbm, srca_hi, dsta_hbm, sa_hbm, src_v, dst_v, bufs_v,
               zbuf_v, acc_sh, sem_g, sem_s, c, s, tid, False)
    _seg_phase(0, xu_hbm, srcu_lo, dstu_hbm, su_hbm, src_v, dst_v, bufs_v,
               zbuf_v, acc_sh, sem_g, sem_s, c, s, tid, False)
    _seg_phase(1, xu_hbm, srcu_hi, dstu_hbm, su_hbm, src_v, dst_v, bufs_v,
               zbuf_v, acc_sh, sem_g, sem_s, c, s, tid, False)


@functools.partial(
    pl.kernel,
    mesh=_mesh,
    out_type=jax.ShapeDtypeStruct((NW, ACC_ROWS), jnp.float32),
    scratch_types=[
        pltpu.VMEM((CPT, CHUNK), jnp.int32),
        pltpu.VMEM((ACC_ROWS,), jnp.float32),
    ],
    compiler_params=_sc_params_nl,
)
def _cnt_sc(dstd_hbm, tok_hbm, cnt_hbm, dst_v, cnt_v):
    del tok_hbm
    c = lax.axis_index("c")
    s = lax.axis_index("s")
    tid = c * NSUB + s

    pltpu.sync_copy(dstd_hbm.at[pl.ds(tid * CPT, CPT)], dst_v)

    def _zc(i, _):
        cnt_v[pl.ds(i * 16, 16)] = jnp.zeros((16,), jnp.float32)
        return 0

    lax.fori_loop(0, ACC_ROWS // 16, _zc, 0)
    ones = jnp.ones((16,), jnp.float32)

    def _cstep(j, _):
        for u in range(CHUNK // 16):
            plsc.addupdate_scatter(cnt_v, [dst_v[j, pl.ds(u * 16, 16)]], ones)
        return 0

    lax.fori_loop(0, CPT, _cstep, 0)
    pltpu.sync_copy(cnt_v, cnt_hbm.at[tid])


@functools.partial(
    pl.kernel,
    mesh=_mesh,
    out_type=jax.ShapeDtypeStruct((2, NCORES, ACC_ROWS, D2), jnp.float32),
    scratch_types=[
        pltpu.VMEM((CPT, CHUNK), jnp.int32),
        pltpu.VMEM((CPT, CHUNK), jnp.int32),
        pltpu.VMEM((NBUF, CHUNK, D2), jnp.float32),
        pltpu.VMEM((QZ, D2), jnp.float32),
        pltpu.VMEM_SHARED((ACC_ROWS, D2), jnp.float32),
        pltpu.SemaphoreType.DMA,
        pltpu.SemaphoreType.DMA,
    ],
    compiler_params=_sc_params,
)
def _down_sc(x_hbm, src_lo, src_hi, dst_hbm, tok_hbm, out_hbm,
             src_v, dst_v, bufs_v, zbuf_v, acc_sh, sem_g, sem_s):
    del tok_hbm
    c = lax.axis_index("c")
    s = lax.axis_index("s")
    tid = c * NSUB + s

    _zero_vmem(zbuf_v, QZ, D2)
    _seg_phase(0, x_hbm, src_lo, dst_hbm, out_hbm, src_v, dst_v, bufs_v,
               zbuf_v, acc_sh, sem_g, sem_s, c, s, tid, True)
    _seg_phase(1, x_hbm, src_hi, dst_hbm, out_hbm, src_v, dst_v, bufs_v,
               zbuf_v, acc_sh, sem_g, sem_s, c, s, tid, False)


BLK = 1024


def _halfdot(lo, hi, w_ref):
    a = jnp.dot(lo, w_ref[0:D2, :], preferred_element_type=jnp.float32)
    return a + jnp.dot(hi, w_ref[D2:D, :], preferred_element_type=jnp.float32)


def _layer_body(is_mean, sd_ref, sa_ref, su_ref, h_ref, cnt_ref,
                w0_ref, w1_ref, w3_ref, wr_ref, b_ref, out_ref):
    sdl = sd_ref[0, 0] + sd_ref[0, 1]
    sdh = sd_ref[1, 0] + sd_ref[1, 1]
    if is_mean:
        cnt = jnp.sum(cnt_ref[...], axis=0)
        inv = (1.0 / jnp.maximum(cnt, 1.0))[:, None]
        sdl = sdl * inv
        sdh = sdh * inv
    acc = _halfdot(sdl, sdh, w0_ref)
    acc = acc + _halfdot(sa_ref[0, 0] + sa_ref[0, 1],
                         sa_ref[1, 0] + sa_ref[1, 1], w1_ref)
    acc = acc + _halfdot(su_ref[0, 0] + su_ref[0, 1],
                         su_ref[1, 0] + su_ref[1, 1], w3_ref)
    acc = acc + jnp.dot(h_ref[...], wr_ref[...], preferred_element_type=jnp.float32)
    out_ref[...] = jnp.maximum(acc + b_ref[...], 0.0)


def _make_layer_tc(is_mean):
    part = pl.BlockSpec((2, NCORES, BLK, D2), lambda i: (0, 0, i, 0))
    full = pl.BlockSpec((D, D), lambda i: (0, 0))
    return pl.pallas_call(
        functools.partial(_layer_body, is_mean),
        grid=(ACC_ROWS // BLK,),
        in_specs=[
            part, part, part,
            pl.BlockSpec((BLK, D), lambda i: (i, 0)),
            pl.BlockSpec((NW, BLK), lambda i: (0, i)),
            full, full, full, full,
            pl.BlockSpec((1, D), lambda i: (0, 0)),
        ],
        out_specs=pl.BlockSpec((BLK, D), lambda i: (i, 0)),
        out_shape=jax.ShapeDtypeStruct((ACC_ROWS, D), jnp.float32),
    )


_layer_tc_sum = _make_layer_tc(False)
_layer_tc_mean = _make_layer_tc(True)


def _final_body(h_ref, wf_ref, bf_ref, out_ref):
    logits = jnp.dot(h_ref[...], wf_ref[...],
                     preferred_element_type=jnp.float32) + bf_ref[...]
    m = jnp.max(logits, axis=1, keepdims=True)
    e = jnp.exp(logits - m)
    out_ref[...] = e / jnp.sum(e, axis=1, keepdims=True)


_final_tc = pl.pallas_call(
    _final_body,
    grid=(ACC_ROWS // BLK,),
    in_specs=[
        pl.BlockSpec((BLK, D), lambda i: (i, 0)),
        pl.BlockSpec((D, OUT), lambda i: (0, 0)),
        pl.BlockSpec((1, OUT), lambda i: (0, 0)),
    ],
    out_specs=pl.BlockSpec((BLK, OUT), lambda i: (i, 0)),
    out_shape=jax.ShapeDtypeStruct((ACC_ROWS, OUT), jnp.float32),
)


def _pad_edges(ei):
    src = jnp.concatenate(
        [ei[0].astype(jnp.int32), jnp.zeros((EPAD - E,), jnp.int32)])
    dst = jnp.concatenate(
        [ei[1].astype(jnp.int32), jnp.full((EPAD - E,), N, jnp.int32)])
    return ((2 * src).reshape(NCHUNK, CHUNK),
            (2 * src + 1).reshape(NCHUNK, CHUNK),
            dst.reshape(NCHUNK, CHUNK))


def kernel(x_sub, x_agr, x_urb, ei_down, ei_agr2sub, ei_sub2agr, ei_urb2sub,
           ei_sub2urb, Wl, bl, Wr, Wf, bf):
    srcd_lo, srcd_hi, dst_d = _pad_edges(ei_down)
    srca_lo, srca_hi, dst_a = _pad_edges(ei_agr2sub)
    srcu_lo, srcu_hi, dst_u = _pad_edges(ei_urb2sub)

    sa, su = _invariants_sc(
        x_agr.reshape(2 * N, D2), x_urb.reshape(2 * N, D2),
        srca_lo, srca_hi, dst_a, srcu_lo, srcu_hi, dst_u)
    cnt = _cnt_sc(dst_d, lax.slice(su.reshape(-1), (0,), (8,)))

    h = jnp.pad(x_sub, ((0, ACC_ROWS - N), (0, 0)))
    tok = lax.slice(cnt[0], (0,), (8,))
    for l in range(3):
        sd = _down_sc(h.reshape(2 * ACC_ROWS, D2), srcd_lo, srcd_hi, dst_d, tok)
        tok = lax.slice(sd.reshape(-1), (0,), (8,))
        wr = Wr[l, 0] + Wr[l, 1] + Wr[l, 3]
        b = (bl[l, 0] + bl[l, 1] + bl[l, 3]).reshape(1, D)
        layer = _layer_tc_sum if l == 0 else _layer_tc_mean
        h = layer(sd, sa, su, h, cnt, Wl[l, 0], Wl[l, 1], Wl[l, 3], wr, b)

    return _final_tc(h, Wf, bf.reshape(1, OUT))[:N]


# speedup vs baseline: 2.7920x; 1.0270x over previous
import functools

import jax
import jax.numpy as jnp
from jax import lax
from jax.experimental import pallas as pl
from jax.experimental.pallas import tpu as pltpu
from jax.experimental.pallas import tpu_sc as plsc

N = 10000
E = 160000
D = 128
D2 = 64
OUT = 8
NCORES = 2
NSUB = 16
NW = NCORES * NSUB
CHUNK = 128
EPAD = 163840
NCHUNK = EPAD // CHUNK
CPT = NCHUNK // NW
ACC_ROWS = 10240
ZROWS = ACC_ROWS // NSUB
QZ = ZROWS // 4
NBUF = 8
AHEAD = 6

_mesh = plsc.VectorSubcoreMesh(core_axis_name="c", subcore_axis_name="s")
_sc_params = pltpu.CompilerParams(use_tc_tiling_on_sc=False)
_sc_params_nl = pltpu.CompilerParams(use_tc_tiling_on_sc=False,
                                     needs_layout_passes=False)


def _zero_vmem(ref, rows, width):
    def _zrow(r, _):
        def _zlane(j, _):
            ref[r, pl.ds(j * 16, 16)] = jnp.zeros((16,), jnp.float32)
            return 0
        return lax.fori_loop(0, width // 16, _zlane, 0)
    lax.fori_loop(0, rows, _zrow, 0)


def _zero_acc(s, zbuf_v, acc_sh):
    for q in range(4):
        pltpu.sync_copy(zbuf_v, acc_sh.at[pl.ds(s * ZROWS + q * QZ, QZ)])


def _accumulate(x_hbm, src_v, dst_v, bufs_v, acc_sh, sem_g, sem_s):
    for p in range(AHEAD):
        pltpu.async_copy(x_hbm.at[src_v.at[p]], bufs_v.at[p], sem_g)

    def _step(j, _):
        b = lax.rem(j, NBUF)
        pltpu.make_async_copy(x_hbm.at[src_v.at[0]], bufs_v.at[0], sem_g).wait()
        pltpu.async_copy(bufs_v.at[b], acc_sh.at[dst_v.at[j]], sem_s, add=True)

        @pl.when(j >= NBUF - AHEAD)
        def _():
            pltpu.make_async_copy(
                bufs_v.at[0], acc_sh.at[dst_v.at[0]], sem_s).wait()

        @pl.when(j + AHEAD < CPT)
        def _():
            b2 = lax.rem(j + AHEAD, NBUF)
            pltpu.async_copy(x_hbm.at[src_v.at[j + AHEAD]], bufs_v.at[b2], sem_g)

        return 0

    lax.fori_loop(0, CPT, _step, 0)
    for p in range(NBUF - AHEAD):
        pltpu.make_async_copy(bufs_v.at[0], acc_sh.at[dst_v.at[0]], sem_s).wait()


def _seg_phase(k, x_hbm, src_hbm, dst_hbm, out_hbm, src_v, dst_v, bufs_v,
               zbuf_v, acc_sh, sem_g, sem_s, c, s, tid, first):
    pltpu.sync_copy(src_hbm.at[pl.ds(tid * CPT, CPT)], src_v)
    pltpu.sync_copy(dst_hbm.at[pl.ds(tid * CPT, CPT)], dst_v)
    if not first:
        plsc.subcore_barrier()
    _zero_acc(s, zbuf_v, acc_sh)
    plsc.subcore_barrier()
    _accumulate(x_hbm, src_v, dst_v, bufs_v, acc_sh, sem_g, sem_s)
    plsc.subcore_barrier()
    r0 = s * ZROWS
    pltpu.sync_copy(acc_sh.at[pl.ds(r0, ZROWS)],
                    out_hbm.at[k].at[c].at[pl.ds(r0, ZROWS)])


@functools.partial(
    pl.kernel,
    mesh=_mesh,
    out_type=(
        jax.ShapeDtypeStruct((2, NCORES, ACC_ROWS, D2), jnp.float32),
        jax.ShapeDtypeStruct((2, NCORES, ACC_ROWS, D2), jnp.float32),
    ),
    scratch_types=[
        pltpu.VMEM((CPT, CHUNK), jnp.int32),
        pltpu.VMEM((CPT, CHUNK), jnp.int32),
        pltpu.VMEM((NBUF, CHUNK, D2), jnp.float32),
        pltpu.VMEM((QZ, D2), jnp.float32),
        pltpu.VMEM_SHARED((ACC_ROWS, D2), jnp.float32),
        pltpu.SemaphoreType.DMA,
        pltpu.SemaphoreType.DMA,
    ],
    compiler_params=_sc_params,
)
def _invariants_sc(xa_hbm, xu_hbm, srca_lo, srca_hi, dsta_hbm,
                   srcu_lo, srcu_hi, dstu_hbm,
                   sa_hbm, su_hbm,
                   src_v, dst_v, bufs_v, zbuf_v, acc_sh, sem_g, sem_s):
    c = lax.axis_index("c")
    s = lax.axis_index("s")
    tid = c * NSUB + s

    _zero_vmem(zbuf_v, QZ, D2)

    _seg_phase(0, xa_hbm, srca_lo, dsta_hbm, sa_hbm, src_v, dst_v, bufs_v,
               zbuf_v, acc_sh, sem_g, sem_s, c, s, tid, True)
    _seg_phase(1, xa_hbm, srca_hi, dsta_hbm, sa_hbm, src_v, dst_v, bufs_v,
               zbuf_v, acc_sh, sem_g, sem_s, c, s, tid, False)
    _seg_phase(0, xu_hbm, srcu_lo, dstu_hbm, su_hbm, src_v, dst_v, bufs_v,
               zbuf_v, acc_sh, sem_g, sem_s, c, s, tid, False)
    _seg_phase(1, xu_hbm, srcu_hi, dstu_hbm, su_hbm, src_v, dst_v, bufs_v,
               zbuf_v, acc_sh, sem_g, sem_s, c, s, tid, False)


@functools.partial(
    pl.kernel,
    mesh=_mesh,
    out_type=jax.ShapeDtypeStruct((NW, ACC_ROWS), jnp.float32),
    scratch_types=[
        pltpu.VMEM((CPT, CHUNK), jnp.int32),
        pltpu.VMEM((ACC_ROWS,), jnp.float32),
    ],
    compiler_params=_sc_params_nl,
)
def _cnt_sc(dstd_hbm, tok_hbm, cnt_hbm, dst_v, cnt_v):
    del tok_hbm
    c = lax.axis_index("c")
    s = lax.axis_index("s")
    tid = c * NSUB + s

    pltpu.sync_copy(dstd_hbm.at[pl.ds(tid * CPT, CPT)], dst_v)

    def _zc(i, _):
        cnt_v[pl.ds(i * 16, 16)] = jnp.zeros((16,), jnp.float32)
        return 0

    lax.fori_loop(0, ACC_ROWS // 16, _zc, 0)
    ones = jnp.ones((16,), jnp.float32)

    def _cstep(j, _):
        for u in range(CHUNK // 16):
            plsc.addupdate_scatter(cnt_v, [dst_v[j, pl.ds(u * 16, 16)]], ones)
        return 0

    lax.fori_loop(0, CPT, _cstep, 0)
    pltpu.sync_copy(cnt_v, cnt_hbm.at[tid])


@functools.partial(
    pl.kernel,
    mesh=_mesh,
    out_type=jax.ShapeDtypeStruct((2, NCORES, ACC_ROWS, D2), jnp.float32),
    scratch_types=[
        pltpu.VMEM((CPT, CHUNK), jnp.int32),
        pltpu.VMEM((CPT, CHUNK), jnp.int32),
        pltpu.VMEM((NBUF, CHUNK, D2), jnp.float32),
        pltpu.VMEM((QZ, D2), jnp.float32),
        pltpu.VMEM_SHARED((ACC_ROWS, D2), jnp.float32),
        pltpu.SemaphoreType.DMA,
        pltpu.SemaphoreType.DMA,
    ],
    compiler_params=_sc_params,
)
def _down_sc(x_hbm, src_lo, src_hi, dst_hbm, tok_hbm, out_hbm,
             src_v, dst_v, bufs_v, zbuf_v, acc_sh, sem_g, sem_s):
    del tok_hbm
    c = lax.axis_index("c")
    s = lax.axis_index("s")
    tid = c * NSUB + s

    _zero_vmem(zbuf_v, QZ, D2)
    _seg_phase(0, x_hbm, src_lo, dst_hbm, out_hbm, src_v, dst_v, bufs_v,
               zbuf_v, acc_sh, sem_g, sem_s, c, s, tid, True)
    _seg_phase(1, x_hbm, src_hi, dst_hbm, out_hbm, src_v, dst_v, bufs_v,
               zbuf_v, acc_sh, sem_g, sem_s, c, s, tid, False)


BLK = 1024


def _halfdot(lo, hi, w_ref):
    a = jnp.dot(lo, w_ref[0:D2, :], preferred_element_type=jnp.float32)
    return a + jnp.dot(hi, w_ref[D2:D, :], preferred_element_type=jnp.float32)


def _layer_body(is_mean, sd_ref, sa_ref, su_ref, h_ref, cnt_ref,
                w0_ref, w1_ref, w3_ref, wr_ref, b_ref, out_ref):
    sdl = sd_ref[0, 0] + sd_ref[0, 1]
    sdh = sd_ref[1, 0] + sd_ref[1, 1]
    if is_mean:
        cnt = jnp.sum(cnt_ref[...], axis=0)
        inv = (1.0 / jnp.maximum(cnt, 1.0))[:, None]
        sdl = sdl * inv
        sdh = sdh * inv
    acc = _halfdot(sdl, sdh, w0_ref)
    acc = acc + _halfdot(sa_ref[0, 0] + sa_ref[0, 1],
                         sa_ref[1, 0] + sa_ref[1, 1], w1_ref)
    acc = acc + _halfdot(su_ref[0, 0] + su_ref[0, 1],
                         su_ref[1, 0] + su_ref[1, 1], w3_ref)
    acc = acc + jnp.dot(h_ref[...], wr_ref[...], preferred_element_type=jnp.float32)
    out_ref[...] = jnp.maximum(acc + b_ref[...], 0.0)


def _make_layer_tc(is_mean):
    part = pl.BlockSpec((2, NCORES, BLK, D2), lambda i: (0, 0, i, 0))
    full = pl.BlockSpec((D, D), lambda i: (0, 0))
    return pl.pallas_call(
        functools.partial(_layer_body, is_mean),
        grid=(ACC_ROWS // BLK,),
        in_specs=[
            part, part, part,
            pl.BlockSpec((BLK, D), lambda i: (i, 0)),
            pl.BlockSpec((NW, BLK), lambda i: (0, i)),
            full, full, full, full,
            pl.BlockSpec((1, D), lambda i: (0, 0)),
        ],
        out_specs=pl.BlockSpec((BLK, D), lambda i: (i, 0)),
        out_shape=jax.ShapeDtypeStruct((ACC_ROWS, D), jnp.float32),
    )


_layer_tc_sum = _make_layer_tc(False)
_layer_tc_mean = _make_layer_tc(True)


def _final_body(h_ref, wf_ref, bf_ref, out_ref):
    logits = jnp.dot(h_ref[...], wf_ref[...],
                     preferred_element_type=jnp.float32) + bf_ref[...]
    m = jnp.max(logits, axis=1, keepdims=True)
    e = jnp.exp(logits - m)
    out_ref[...] = e / jnp.sum(e, axis=1, keepdims=True)


_final_tc = pl.pallas_call(
    _final_body,
    grid=(ACC_ROWS // BLK,),
    in_specs=[
        pl.BlockSpec((BLK, D), lambda i: (i, 0)),
        pl.BlockSpec((D, OUT), lambda i: (0, 0)),
        pl.BlockSpec((1, OUT), lambda i: (0, 0)),
    ],
    out_specs=pl.BlockSpec((BLK, OUT), lambda i: (i, 0)),
    out_shape=jax.ShapeDtypeStruct((ACC_ROWS, OUT), jnp.float32),
)


def _pad_edges(ei):
    src = jnp.concatenate(
        [ei[0].astype(jnp.int32), jnp.zeros((EPAD - E,), jnp.int32)])
    dst = jnp.concatenate(
        [ei[1].astype(jnp.int32), jnp.full((EPAD - E,), N, jnp.int32)])
    return ((2 * src).reshape(NCHUNK, CHUNK),
            (2 * src + 1).reshape(NCHUNK, CHUNK),
            dst.reshape(NCHUNK, CHUNK))


def kernel(x_sub, x_agr, x_urb, ei_down, ei_agr2sub, ei_sub2agr, ei_urb2sub,
           ei_sub2urb, Wl, bl, Wr, Wf, bf):
    srcd_lo, srcd_hi, dst_d = _pad_edges(ei_down)
    srca_lo, srca_hi, dst_a = _pad_edges(ei_agr2sub)
    srcu_lo, srcu_hi, dst_u = _pad_edges(ei_urb2sub)

    sa, su = _invariants_sc(
        x_agr.reshape(2 * N, D2), x_urb.reshape(2 * N, D2),
        srca_lo, srca_hi, dst_a, srcu_lo, srcu_hi, dst_u)
    cnt = _cnt_sc(dst_d, lax.slice(su.reshape(-1), (0,), (8,)))

    h = jnp.pad(x_sub, ((0, ACC_ROWS - N), (0, 0)))
    tok = lax.slice(cnt[0], (0,), (8,))
    for l in range(3):
        sd = _down_sc(h.reshape(2 * ACC_ROWS, D2), srcd_lo, srcd_hi, dst_d, tok)
        tok = lax.slice(sd.reshape(-1), (0,), (8,))
        wr = Wr[l, 0] + Wr[l, 1] + Wr[l, 3]
        b = (bl[l, 0] + bl[l, 1] + bl[l, 3]).reshape(1, D)
        layer = _layer_tc_sum if l == 0 else _layer_tc_mean
        h = layer(sd, sa, su, h, cnt, Wl[l, 0], Wl[l, 1], Wl[l, 3], wr, b)

    return _final_tc(h, Wf, bf.reshape(1, OUT))[:N]
